# Initial kernel scaffold; baseline (speedup 1.0000x reference)
import jax
import jax.numpy as jnp
from jax.experimental import pallas as pl


def kernel(x, edge_index, params):
    raise NotImplementedError("write your pallas kernel here")


# speedup vs baseline: 6.8526x; 6.8526x over previous
import functools

import jax
import jax.numpy as jnp
from jax import lax
from jax.experimental import pallas as pl
from jax.experimental.pallas import tpu as pltpu
from jax.experimental.pallas import tpu_sc as plsc

N = 10000
E = 160000
D_IN = 256
FH1 = 512
FH2 = 256
GH = 128
LAT = 64

NTILES = 32
NSUB = 16
BLK = 128
E_PAD = 163840
NB_EDGE = E_PAD // (NTILES * BLK)
NB_COL = E_PAD // (NSUB * BLK)
NACC = 10112
RPT = NACC // NSUB

_HIGH = jax.lax.Precision.HIGHEST


def _mesh():
    return plsc.VectorSubcoreMesh(core_axis_name="c", subcore_axis_name="s")


_SC_PARAMS = pltpu.CompilerParams(use_tc_tiling_on_sc=False)


def _deg_body(dst_hbm, ones_hbm, zeros_hbm, out_hbm, idx_v, ones_v, sem, acc_sh):
    c = lax.axis_index("c")
    s = lax.axis_index("s")
    w = c * NSUB + s
    pltpu.sync_copy(dst_hbm.at[w], idx_v)
    pltpu.sync_copy(ones_hbm, ones_v)
    pltpu.sync_copy(zeros_hbm.at[pl.ds(s * RPT, RPT)], acc_sh.at[pl.ds(s * RPT, RPT)])
    plsc.subcore_barrier()

    def blk(j, carry):
        pltpu.async_copy(ones_v, acc_sh.at[idx_v.at[j]], sem, add=True).wait()
        return carry

    lax.fori_loop(0, NB_EDGE, blk, 0)
    plsc.subcore_barrier()
    pltpu.sync_copy(acc_sh.at[pl.ds(s * RPT, RPT)],
                    out_hbm.at[c].at[pl.ds(s * RPT, RPT)])


def _deg_kernel():
    return pl.kernel(
        _deg_body,
        out_type=jax.ShapeDtypeStruct((2, NACC, 16), jnp.float32),
        mesh=_mesh(),
        compiler_params=_SC_PARAMS,
        scratch_types=[
            pltpu.VMEM((NB_EDGE, BLK), jnp.int32),
            pltpu.VMEM((BLK, 16), jnp.float32),
            pltpu.SemaphoreType.DMA,
            pltpu.VMEM_SHARED((NACC, 16), jnp.float32),
        ],
    )


def _prop_body(F, src_hbm, dst_hbm, mt_hbm, zeros_hbm, out_hbm,
               isrc_v, idst_v, rows_v, semg, sems, acc_sh):
    c = lax.axis_index("c")
    s = lax.axis_index("s")
    w = c * NSUB + s
    pltpu.sync_copy(src_hbm.at[w], isrc_v)
    pltpu.sync_copy(dst_hbm.at[w], idst_v)
    pltpu.sync_copy(zeros_hbm.at[pl.ds(s * RPT, RPT)], acc_sh.at[pl.ds(s * RPT, RPT)])
    plsc.subcore_barrier()

    def blk(j, carry):
        pltpu.async_copy(mt_hbm.at[isrc_v.at[j]], rows_v, semg).wait()
        pltpu.async_copy(rows_v, acc_sh.at[idst_v.at[j]], sems, add=True).wait()
        return carry

    lax.fori_loop(0, NB_EDGE, blk, 0)
    plsc.subcore_barrier()
    pltpu.sync_copy(acc_sh.at[pl.ds(s * RPT, RPT)],
                    out_hbm.at[c].at[pl.ds(s * RPT, RPT)])


def _prop_kernel(F):
    return pl.kernel(
        functools.partial(_prop_body, F),
        out_type=jax.ShapeDtypeStruct((2, NACC, F), jnp.float32),
        mesh=_mesh(),
        compiler_params=_SC_PARAMS,
        scratch_types=[
            pltpu.VMEM((NB_EDGE, BLK), jnp.int32),
            pltpu.VMEM((NB_EDGE, BLK), jnp.int32),
            pltpu.VMEM((BLK, F), jnp.float32),
            pltpu.SemaphoreType.DMA,
            pltpu.SemaphoreType.DMA,
            pltpu.VMEM_SHARED((NACC, F), jnp.float32),
        ],
    )


def _prop_col_body(src_hbm, dst_hbm, mt_hbm, zeros_hbm, out_hbm,
                   isrc_v, idst_v, rows_v, semg, sems, acc_sh):
    c = lax.axis_index("c")
    s = lax.axis_index("s")
    pltpu.sync_copy(src_hbm.at[s], isrc_v)
    pltpu.sync_copy(dst_hbm.at[s], idst_v)
    pltpu.sync_copy(zeros_hbm.at[pl.ds(s * RPT, RPT)], acc_sh.at[pl.ds(s * RPT, RPT)])
    plsc.subcore_barrier()

    def blk(j, carry):
        pltpu.async_copy(mt_hbm.at[c].at[isrc_v.at[j]], rows_v, semg).wait()
        pltpu.async_copy(rows_v, acc_sh.at[idst_v.at[j]], sems, add=True).wait()
        return carry

    lax.fori_loop(0, NB_COL, blk, 0)
    plsc.subcore_barrier()
    pltpu.sync_copy(acc_sh.at[pl.ds(s * RPT, RPT)],
                    out_hbm.at[c].at[pl.ds(s * RPT, RPT)])


def _prop_col_kernel():
    return pl.kernel(
        _prop_col_body,
        out_type=jax.ShapeDtypeStruct((2, NACC, 128), jnp.float32),
        mesh=_mesh(),
        compiler_params=_SC_PARAMS,
        scratch_types=[
            pltpu.VMEM((NB_COL, BLK), jnp.int32),
            pltpu.VMEM((NB_COL, BLK), jnp.int32),
            pltpu.VMEM((BLK, 128), jnp.float32),
            pltpu.SemaphoreType.DMA,
            pltpu.SemaphoreType.DMA,
            pltpu.VMEM_SHARED((NACC, 128), jnp.float32),
        ],
    )


BR = 1000


def _dinv_from(deg_ref):
    d = deg_ref[0, :, 0] + deg_ref[1, :, 0] + 1.0
    return lax.rsqrt(d)


def _elu(h):
    return jnp.where(h > 0.0, h, jnp.exp(h) - 1.0)


def _stageA_body(x_ref, w1_ref, a1_ref, c1_ref, w2_ref, a2_ref, c2_ref,
                 g1_ref, deg_ref, m1_ref):
    dinv = _dinv_from(deg_ref)
    h = jnp.dot(x_ref[...], w1_ref[...], precision=_HIGH)
    h = _elu(h * a1_ref[...] + c1_ref[...])
    h = jnp.dot(h, w2_ref[...], precision=_HIGH)
    h = _elu(h * a2_ref[...] + c2_ref[...])
    m = jnp.dot(h, g1_ref[...], precision=_HIGH)
    m1_ref[...] = m * dinv[:, None]


def _stageB_body(p_ref, m1_ref, deg_ref, ag_ref, cg_ref, g2_ref, m2_ref):
    dinv = _dinv_from(deg_ref)
    ssum = p_ref[0] + p_ref[1] + m1_ref[...]
    h3 = jnp.maximum(ssum * dinv[:, None] * ag_ref[...] + cg_ref[...], 0.0)
    m2_ref[...] = jnp.dot(h3, g2_ref[...], precision=_HIGH) * dinv[:, None]


def _stageC_body(q_ref, m2_ref, deg_ref, ag_ref, cg_ref, e2d_ref, dec_ref, m3_ref):
    dinv = _dinv_from(deg_ref)
    ssum = q_ref[0] + q_ref[1] + m2_ref[...]
    h4 = ssum * dinv[:, None] * ag_ref[...] + cg_ref[...]
    h5 = jnp.dot(h4, e2d_ref[...], precision=_HIGH)
    m3 = jnp.dot(h5, dec_ref[...], precision=_HIGH) * dinv[:, None]
    m3_ref[0] = m3[:, :128]
    m3_ref[1] = m3[:, 128:]


def _stageD_body(r_ref, m3_ref, deg_ref, ad_ref, cd_ref, out_ref):
    dinv = _dinv_from(deg_ref)
    s0 = (r_ref[0] + m3_ref[0]) * dinv[:, None]
    s1 = (r_ref[1] + m3_ref[1]) * dinv[:, None]
    out_ref[...] = jnp.concatenate(
        [s0 * ad_ref[0, 0] + cd_ref[0, 0], s1 * ad_ref[0, 1] + cd_ref[0, 1]], axis=1)


def _row_spec(shape2):
    return pl.BlockSpec((BR,) + shape2[1:], lambda i: (i,) + (0,) * (len(shape2) - 1))


def _full_spec(shape):
    return pl.BlockSpec(shape, lambda i: (0,) * len(shape))


def _part_spec(F):
    return pl.BlockSpec((2, BR, F), lambda i: (0, i, 0))


def kernel(x, edge_index, params):
    p = params
    f32 = jnp.float32

    src = edge_index[0]
    dst = edge_index[1]
    pad = E_PAD - E
    src_p = jnp.concatenate([src, jnp.zeros((pad,), jnp.int32)])
    dst_p = jnp.concatenate([dst, jnp.full((pad,), N, jnp.int32)])
    src_e = src_p.reshape(NTILES, NB_EDGE, BLK)
    dst_e = dst_p.reshape(NTILES, NB_EDGE, BLK)
    src_c = src_p.reshape(NSUB, NB_COL, BLK)
    dst_c = dst_p.reshape(NSUB, NB_COL, BLK)

    ones16 = jnp.ones((BLK, 16), f32)
    z16 = jnp.zeros((NACC, 16), f32)
    z64 = jnp.zeros((NACC, 64), f32)
    z128 = jnp.zeros((NACC, 128), f32)

    def row(v):
        return v.reshape(1, -1).astype(f32)

    a1 = p["bn1_g"] / jnp.sqrt(1.0 + 1e-3)
    c1 = p["enc1_b"] * a1 + p["bn1_b"]
    a2 = p["bn2_g"] / jnp.sqrt(1.0 + 1e-3)
    c2 = p["enc2_b"] * a2 + p["bn2_b"]
    ag1 = p["gc1bn_g"] / jnp.sqrt(1.0 + 1e-5)
    cg1 = p["gc1_b"] * ag1 + p["gc1bn_b"]
    ag2 = p["gc2bn_g"] / jnp.sqrt(1.0 + 1e-5)
    cg2 = p["gc2_b"] * ag2 + p["gc2bn_b"]
    ad = p["decbn_g"] / jnp.sqrt(1.0 + 1e-5)
    cd = p["dec_b"] * ad + p["decbn_b"]
    ad2 = ad.reshape(1, 2, 128)
    cd2 = cd.reshape(1, 2, 128)

    w1t = p["enc1_W"].T
    w2t = p["enc2_W"].T
    g1t = p["gc1_W"].T
    g2t = p["gc2_W"].T
    e2dt = p["e2d_W"].T
    dect = p["dec_W"].T

    deg = _deg_kernel()(dst_e, ones16, z16)

    m1 = pl.pallas_call(
        _stageA_body,
        grid=(N // BR,),
        in_specs=[
            _row_spec((N, D_IN)),
            _full_spec((D_IN, FH1)), _full_spec((1, FH1)), _full_spec((1, FH1)),
            _full_spec((FH1, FH2)), _full_spec((1, FH2)), _full_spec((1, FH2)),
            _full_spec((FH2, GH)),
            _part_spec(16),
        ],
        out_specs=_row_spec((N, GH)),
        out_shape=jax.ShapeDtypeStruct((N, GH), f32),
    )(x, w1t, row(a1), row(c1), w2t, row(a2), row(c2), g1t, deg)

    p1 = _prop_kernel(GH)(src_e, dst_e, m1, z128)

    m2 = pl.pallas_call(
        _stageB_body,
        grid=(N // BR,),
        in_specs=[
            _part_spec(GH), _row_spec((N, GH)), _part_spec(16),
            _full_spec((1, GH)), _full_spec((1, GH)), _full_spec((GH, LAT)),
        ],
        out_specs=_row_spec((N, LAT)),
        out_shape=jax.ShapeDtypeStruct((N, LAT), f32),
    )(p1, m1, deg, row(ag1), row(cg1), g2t)

    p2 = _prop_kernel(LAT)(src_e, dst_e, m2, z64)

    m3 = pl.pallas_call(
        _stageC_body,
        grid=(N // BR,),
        in_specs=[
            _part_spec(LAT), _row_spec((N, LAT)), _part_spec(16),
            _full_spec((1, LAT)), _full_spec((1, LAT)),
            _full_spec((LAT, LAT)), _full_spec((LAT, D_IN)),
        ],
        out_specs=pl.BlockSpec((2, BR, 128), lambda i: (0, i, 0)),
        out_shape=jax.ShapeDtypeStruct((2, N, 128), f32),
    )(p2, m2, deg, row(ag2), row(cg2), e2dt, dect)

    p3 = _prop_col_kernel()(src_c, dst_c, m3, z128)

    recon = pl.pallas_call(
        _stageD_body,
        grid=(N // BR,),
        in_specs=[
            _part_spec(128),
            pl.BlockSpec((2, BR, 128), lambda i: (0, i, 0)),
            _part_spec(16),
            _full_spec((1, 2, 128)), _full_spec((1, 2, 128)),
        ],
        out_specs=_row_spec((N, D_IN)),
        out_shape=jax.ShapeDtypeStruct((N, D_IN), f32),
    )(p3, m3, deg, ad2, cd2)

    return recon


# speedup vs baseline: 7.4372x; 1.0853x over previous
import functools

import jax
import jax.numpy as jnp
from jax import lax
from jax.experimental import pallas as pl
from jax.experimental.pallas import tpu as pltpu
from jax.experimental.pallas import tpu_sc as plsc

N = 10000
E = 160000
D_IN = 256
FH1 = 512
FH2 = 256
GH = 128
LAT = 64

NTILES = 32
NSUB = 16
BLK = 128
E_PAD = 163840
NB_EDGE = E_PAD // (NTILES * BLK)
NB_COL = E_PAD // (NSUB * BLK)
NACC = 10112
RPT = NACC // NSUB

_HIGH = jax.lax.Precision.HIGHEST


def _mesh():
    return plsc.VectorSubcoreMesh(core_axis_name="c", subcore_axis_name="s")


_SC_PARAMS = pltpu.CompilerParams(use_tc_tiling_on_sc=False)


def _deg_body(dst_hbm, ones_hbm, zeros_hbm, out_hbm, idx_v, ones_v, sem, acc_sh):
    c = lax.axis_index("c")
    s = lax.axis_index("s")
    w = c * NSUB + s
    pltpu.sync_copy(dst_hbm.at[w], idx_v)
    pltpu.sync_copy(ones_hbm, ones_v)
    pltpu.sync_copy(zeros_hbm.at[pl.ds(s * RPT, RPT)], acc_sh.at[pl.ds(s * RPT, RPT)])
    plsc.subcore_barrier()

    def blk(j, carry):
        pltpu.async_copy(ones_v, acc_sh.at[idx_v.at[j]], sem, add=True).wait()
        return carry

    lax.fori_loop(0, NB_EDGE, blk, 0)
    plsc.subcore_barrier()
    pltpu.sync_copy(acc_sh.at[pl.ds(s * RPT, RPT)],
                    out_hbm.at[c].at[pl.ds(s * RPT, RPT)])


def _deg_kernel():
    return pl.kernel(
        _deg_body,
        out_type=jax.ShapeDtypeStruct((2, NACC, 16), jnp.float32),
        mesh=_mesh(),
        compiler_params=_SC_PARAMS,
        scratch_types=[
            pltpu.VMEM((NB_EDGE, BLK), jnp.int32),
            pltpu.VMEM((BLK, 16), jnp.float32),
            pltpu.SemaphoreType.DMA,
            pltpu.VMEM_SHARED((NACC, 16), jnp.float32),
        ],
    )


def _pipelined_scatter(nb, slab_src, slab_dst, gather_table, isrc_r, idst_r,
                       rows, semg, sems, semis, semid, acc_sh):
    def i_start(j, sl):
        pltpu.async_copy(slab_src.at[j], isrc_r.at[sl], semis[sl])
        pltpu.async_copy(slab_dst.at[j], idst_r.at[sl], semid[sl])

    def i_wait(j, sl):
        pltpu.make_async_copy(slab_src.at[j], isrc_r.at[sl], semis[sl]).wait()
        pltpu.make_async_copy(slab_dst.at[j], idst_r.at[sl], semid[sl]).wait()

    def g_start(sl4, b):
        pltpu.async_copy(gather_table(isrc_r.at[sl4]), rows[b], semg[b])

    def g_wait(sl4, b):
        pltpu.make_async_copy(gather_table(isrc_r.at[sl4]), rows[b], semg[b]).wait()

    def s_start(sl4, b):
        pltpu.async_copy(rows[b], acc_sh.at[idst_r.at[sl4]], sems[b], add=True)

    def s_wait(sl4, b):
        pltpu.make_async_copy(rows[b], acc_sh.at[idst_r.at[sl4]], sems[b]).wait()

    for j0 in range(4):
        i_start(j0, j0)
    i_wait(0, 0)
    g_start(0, 0)

    def group(g, carry):
        j0 = g * 4
        for b in range(4):
            j = j0 + b
            g_wait(b, b % 2)
            s_start(b, b % 2)

            @pl.when(j >= 1)
            def _():
                s_wait((b - 1) % 4, (b - 1) % 2)

            @pl.when(jnp.logical_and(j >= 1, j + 3 < nb))
            def _():
                i_start(j + 3, (b + 3) % 4)

            @pl.when(j + 1 < nb)
            def _():
                i_wait(j + 1, (b + 1) % 4)
                g_start((b + 1) % 4, (b + 1) % 2)

        return carry

    lax.fori_loop(0, nb // 4, group, 0)
    s_wait(3, 1)


def _prop_body(F, src_hbm, dst_hbm, mt_hbm, zeros_hbm, out_hbm,
               isrc_r, idst_r, r0, r1,
               sg0, sg1, ss0, ss1,
               si0, si1, si2, si3, sd0, sd1, sd2, sd3, acc_sh):
    c = lax.axis_index("c")
    s = lax.axis_index("s")
    w = c * NSUB + s
    pltpu.sync_copy(zeros_hbm.at[pl.ds(s * RPT, RPT)], acc_sh.at[pl.ds(s * RPT, RPT)])
    plsc.subcore_barrier()
    _pipelined_scatter(NB_EDGE, src_hbm.at[w], dst_hbm.at[w],
                       lambda ix: mt_hbm.at[ix], isrc_r, idst_r,
                       (r0, r1), (sg0, sg1), (ss0, ss1),
                       (si0, si1, si2, si3), (sd0, sd1, sd2, sd3), acc_sh)
    plsc.subcore_barrier()
    pltpu.sync_copy(acc_sh.at[pl.ds(s * RPT, RPT)],
                    out_hbm.at[c].at[pl.ds(s * RPT, RPT)])


def _prop_kernel(F):
    return pl.kernel(
        functools.partial(_prop_body, F),
        out_type=jax.ShapeDtypeStruct((2, NACC, F), jnp.float32),
        mesh=_mesh(),
        compiler_params=_SC_PARAMS,
        scratch_types=_prop_scratch(F),
    )


def _prop_scratch(F):
    return ([pltpu.VMEM((4, BLK), jnp.int32)] * 2
            + [pltpu.VMEM((BLK, F), jnp.float32)] * 2
            + [pltpu.SemaphoreType.DMA] * 12
            + [pltpu.VMEM_SHARED((NACC, F), jnp.float32)])


def _prop_col_body(src_hbm, dst_hbm, mt_hbm, zeros_hbm, out_hbm,
                   isrc_r, idst_r, r0, r1,
                   sg0, sg1, ss0, ss1,
                   si0, si1, si2, si3, sd0, sd1, sd2, sd3, acc_sh):
    c = lax.axis_index("c")
    s = lax.axis_index("s")
    pltpu.sync_copy(zeros_hbm.at[pl.ds(s * RPT, RPT)], acc_sh.at[pl.ds(s * RPT, RPT)])
    plsc.subcore_barrier()
    _pipelined_scatter(NB_COL, src_hbm.at[s], dst_hbm.at[s],
                       lambda ix: mt_hbm.at[c].at[ix], isrc_r, idst_r,
                       (r0, r1), (sg0, sg1), (ss0, ss1),
                       (si0, si1, si2, si3), (sd0, sd1, sd2, sd3), acc_sh)
    plsc.subcore_barrier()
    pltpu.sync_copy(acc_sh.at[pl.ds(s * RPT, RPT)],
                    out_hbm.at[c].at[pl.ds(s * RPT, RPT)])


def _prop_col_kernel():
    return pl.kernel(
        _prop_col_body,
        out_type=jax.ShapeDtypeStruct((2, NACC, 128), jnp.float32),
        mesh=_mesh(),
        compiler_params=_SC_PARAMS,
        scratch_types=_prop_scratch(128),
    )


BR = 1000


def _dinv_from(deg_ref):
    d = deg_ref[0, :, 0] + deg_ref[1, :, 0] + 1.0
    return lax.rsqrt(d)


def _elu(h):
    return jnp.where(h > 0.0, h, jnp.exp(h) - 1.0)


def _stageA_body(x_ref, w1_ref, a1_ref, c1_ref, w2_ref, a2_ref, c2_ref,
                 g1_ref, deg_ref, m1_ref):
    dinv = _dinv_from(deg_ref)
    h = jnp.dot(x_ref[...], w1_ref[...], precision=_HIGH)
    h = _elu(h * a1_ref[...] + c1_ref[...])
    h = jnp.dot(h, w2_ref[...], precision=_HIGH)
    h = _elu(h * a2_ref[...] + c2_ref[...])
    m = jnp.dot(h, g1_ref[...], precision=_HIGH)
    m1_ref[...] = m * dinv[:, None]


def _stageB_body(p_ref, m1_ref, deg_ref, ag_ref, cg_ref, g2_ref, m2_ref):
    dinv = _dinv_from(deg_ref)
    ssum = p_ref[0] + p_ref[1] + m1_ref[...]
    h3 = jnp.maximum(ssum * dinv[:, None] * ag_ref[...] + cg_ref[...], 0.0)
    m2_ref[...] = jnp.dot(h3, g2_ref[...], precision=_HIGH) * dinv[:, None]


def _stageC_body(q_ref, m2_ref, deg_ref, ag_ref, cg_ref, e2d_ref, dec_ref, m3_ref):
    dinv = _dinv_from(deg_ref)
    ssum = q_ref[0] + q_ref[1] + m2_ref[...]
    h4 = ssum * dinv[:, None] * ag_ref[...] + cg_ref[...]
    h5 = jnp.dot(h4, e2d_ref[...], precision=_HIGH)
    m3 = jnp.dot(h5, dec_ref[...], precision=_HIGH) * dinv[:, None]
    m3_ref[0] = m3[:, :128]
    m3_ref[1] = m3[:, 128:]


def _stageD_body(r_ref, m3_ref, deg_ref, ad_ref, cd_ref, out_ref):
    dinv = _dinv_from(deg_ref)
    s0 = (r_ref[0] + m3_ref[0]) * dinv[:, None]
    s1 = (r_ref[1] + m3_ref[1]) * dinv[:, None]
    out_ref[...] = jnp.concatenate(
        [s0 * ad_ref[0, 0] + cd_ref[0, 0], s1 * ad_ref[0, 1] + cd_ref[0, 1]], axis=1)


def _row_spec(shape2):
    return pl.BlockSpec((BR,) + shape2[1:], lambda i: (i,) + (0,) * (len(shape2) - 1))


def _full_spec(shape):
    return pl.BlockSpec(shape, lambda i: (0,) * len(shape))


def _part_spec(F):
    return pl.BlockSpec((2, BR, F), lambda i: (0, i, 0))


def kernel(x, edge_index, params):
    p = params
    f32 = jnp.float32

    src = edge_index[0]
    dst = edge_index[1]
    pad = E_PAD - E
    src_p = jnp.concatenate([src, jnp.zeros((pad,), jnp.int32)])
    dst_p = jnp.concatenate([dst, jnp.full((pad,), N, jnp.int32)])
    src_e = src_p.reshape(NTILES, NB_EDGE, BLK)
    dst_e = dst_p.reshape(NTILES, NB_EDGE, BLK)
    src_c = src_p.reshape(NSUB, NB_COL, BLK)
    dst_c = dst_p.reshape(NSUB, NB_COL, BLK)

    ones16 = jnp.ones((BLK, 16), f32)
    z16 = jnp.zeros((NACC, 16), f32)
    z64 = jnp.zeros((NACC, 64), f32)
    z128 = jnp.zeros((NACC, 128), f32)

    def row(v):
        return v.reshape(1, -1).astype(f32)

    a1 = p["bn1_g"] / jnp.sqrt(1.0 + 1e-3)
    c1 = p["enc1_b"] * a1 + p["bn1_b"]
    a2 = p["bn2_g"] / jnp.sqrt(1.0 + 1e-3)
    c2 = p["enc2_b"] * a2 + p["bn2_b"]
    ag1 = p["gc1bn_g"] / jnp.sqrt(1.0 + 1e-5)
    cg1 = p["gc1_b"] * ag1 + p["gc1bn_b"]
    ag2 = p["gc2bn_g"] / jnp.sqrt(1.0 + 1e-5)
    cg2 = p["gc2_b"] * ag2 + p["gc2bn_b"]
    ad = p["decbn_g"] / jnp.sqrt(1.0 + 1e-5)
    cd = p["dec_b"] * ad + p["decbn_b"]
    ad2 = ad.reshape(1, 2, 128)
    cd2 = cd.reshape(1, 2, 128)

    w1t = p["enc1_W"].T
    w2t = p["enc2_W"].T
    g1t = p["gc1_W"].T
    g2t = p["gc2_W"].T
    e2dt = p["e2d_W"].T
    dect = p["dec_W"].T

    deg = _deg_kernel()(dst_e, ones16, z16)

    m1 = pl.pallas_call(
        _stageA_body,
        grid=(N // BR,),
        in_specs=[
            _row_spec((N, D_IN)),
            _full_spec((D_IN, FH1)), _full_spec((1, FH1)), _full_spec((1, FH1)),
            _full_spec((FH1, FH2)), _full_spec((1, FH2)), _full_spec((1, FH2)),
            _full_spec((FH2, GH)),
            _part_spec(16),
        ],
        out_specs=_row_spec((N, GH)),
        out_shape=jax.ShapeDtypeStruct((N, GH), f32),
    )(x, w1t, row(a1), row(c1), w2t, row(a2), row(c2), g1t, deg)

    p1 = _prop_kernel(GH)(src_e, dst_e, m1, z128)

    m2 = pl.pallas_call(
        _stageB_body,
        grid=(N // BR,),
        in_specs=[
            _part_spec(GH), _row_spec((N, GH)), _part_spec(16),
            _full_spec((1, GH)), _full_spec((1, GH)), _full_spec((GH, LAT)),
        ],
        out_specs=_row_spec((N, LAT)),
        out_shape=jax.ShapeDtypeStruct((N, LAT), f32),
    )(p1, m1, deg, row(ag1), row(cg1), g2t)

    p2 = _prop_kernel(LAT)(src_e, dst_e, m2, z64)

    m3 = pl.pallas_call(
        _stageC_body,
        grid=(N // BR,),
        in_specs=[
            _part_spec(LAT), _row_spec((N, LAT)), _part_spec(16),
            _full_spec((1, LAT)), _full_spec((1, LAT)),
            _full_spec((LAT, LAT)), _full_spec((LAT, D_IN)),
        ],
        out_specs=pl.BlockSpec((2, BR, 128), lambda i: (0, i, 0)),
        out_shape=jax.ShapeDtypeStruct((2, N, 128), f32),
    )(p2, m2, deg, row(ag2), row(cg2), e2dt, dect)

    p3 = _prop_col_kernel()(src_c, dst_c, m3, z128)

    recon = pl.pallas_call(
        _stageD_body,
        grid=(N // BR,),
        in_specs=[
            _part_spec(128),
            pl.BlockSpec((2, BR, 128), lambda i: (0, i, 0)),
            _part_spec(16),
            _full_spec((1, 2, 128)), _full_spec((1, 2, 128)),
        ],
        out_specs=_row_spec((N, D_IN)),
        out_shape=jax.ShapeDtypeStruct((N, D_IN), f32),
    )(p3, m3, deg, ad2, cd2)

    return recon


# speedup vs baseline: 12.2013x; 1.6406x over previous
import functools

import jax
import jax.numpy as jnp
from jax import lax
from jax.experimental import pallas as pl
from jax.experimental.pallas import tpu as pltpu
from jax.experimental.pallas import tpu_sc as plsc

N = 10000
E = 160000
D_IN = 256
FH1 = 512
FH2 = 256
GH = 128
LAT = 64

NTILES = 32
NSUB = 16
BLK = 128
E_PAD = 163840
NB_EDGE = E_PAD // (NTILES * BLK)
NB_COL = E_PAD // (NSUB * BLK)
NACC = 10112
RPT = NACC // NSUB

_HIGH = jax.lax.Precision.HIGHEST


def _mesh():
    return plsc.VectorSubcoreMesh(core_axis_name="c", subcore_axis_name="s")


_SC_PARAMS = pltpu.CompilerParams(use_tc_tiling_on_sc=False)


def _deg_body(dst_hbm, ones_hbm, zeros_hbm, out_hbm, idx_v, ones_v, sem, acc_sh):
    c = lax.axis_index("c")
    s = lax.axis_index("s")
    w = c * NSUB + s
    pltpu.sync_copy(dst_hbm.at[w], idx_v)
    pltpu.sync_copy(ones_hbm, ones_v)
    pltpu.sync_copy(zeros_hbm.at[pl.ds(s * RPT, RPT)], acc_sh.at[pl.ds(s * RPT, RPT)])
    plsc.subcore_barrier()

    def blk(j, carry):
        pltpu.async_copy(ones_v, acc_sh.at[idx_v.at[j]], sem, add=True).wait()
        return carry

    lax.fori_loop(0, NB_EDGE, blk, 0)
    plsc.subcore_barrier()
    pltpu.sync_copy(acc_sh.at[pl.ds(s * RPT, RPT)],
                    out_hbm.at[c].at[pl.ds(s * RPT, RPT)])


def _deg_kernel():
    return pl.kernel(
        _deg_body,
        out_type=jax.ShapeDtypeStruct((2, NACC, 16), jnp.float32),
        mesh=_mesh(),
        compiler_params=_SC_PARAMS,
        scratch_types=[
            pltpu.VMEM((NB_EDGE, BLK), jnp.int32),
            pltpu.VMEM((BLK, 16), jnp.float32),
            pltpu.SemaphoreType.DMA,
            pltpu.VMEM_SHARED((NACC, 16), jnp.float32),
        ],
    )


def _edge_loop(nb, slab_src, slab_dst, table_sh, isrc_r, idst_r,
               rows, semg, sems, semis, semid, acc_sh):
    def i_start(j, sl):
        pltpu.async_copy(slab_src.at[j], isrc_r.at[sl], semis[sl])
        pltpu.async_copy(slab_dst.at[j], idst_r.at[sl], semid[sl])

    def i_wait(j, sl):
        pltpu.make_async_copy(slab_src.at[j], isrc_r.at[sl], semis[sl]).wait()
        pltpu.make_async_copy(slab_dst.at[j], idst_r.at[sl], semid[sl]).wait()

    def g_start(sl4, b):
        pltpu.async_copy(table_sh.at[isrc_r.at[sl4]], rows[b], semg[b])

    def g_wait(sl4, b):
        pltpu.make_async_copy(table_sh.at[isrc_r.at[sl4]], rows[b], semg[b]).wait()

    def s_start(sl4, b):
        pltpu.async_copy(rows[b], acc_sh.at[idst_r.at[sl4]], sems[b], add=True)

    def s_wait(sl4, b):
        pltpu.make_async_copy(rows[b], acc_sh.at[idst_r.at[sl4]], sems[b]).wait()

    for j0 in range(4):
        i_start(j0, j0)
    i_wait(0, 0)
    g_start(0, 0)

    def group(g, carry):
        j0 = g * 4
        for b in range(4):
            j = j0 + b
            g_wait(b, b % 2)
            s_start(b, b % 2)

            @pl.when(j >= 1)
            def _():
                s_wait((b - 1) % 4, (b - 1) % 2)

            @pl.when(jnp.logical_and(j >= 1, j + 3 < nb))
            def _():
                i_start(j + 3, (b + 3) % 4)

            @pl.when(j + 1 < nb)
            def _():
                i_wait(j + 1, (b + 1) % 4)
                g_start((b + 1) % 4, (b + 1) % 2)

        return carry

    lax.fori_loop(0, nb // 4, group, 0)
    s_wait(3, 1)


def _prop_body(NCH, src_hbm, dst_hbm, mt_hbm, zeros_hbm, out_hbm,
               isrc_r, idst_r, r0, r1,
               sg0, sg1, ss0, ss1,
               si0, si1, si2, si3, sd0, sd1, sd2, sd3, table_sh, acc_sh):
    c = lax.axis_index("c")
    s = lax.axis_index("s")
    rows_sl = pl.ds(s * RPT, RPT)

    for p in range(NCH // 2):
        ch = c * (NCH // 2) + p
        pltpu.sync_copy(mt_hbm.at[ch].at[rows_sl], table_sh.at[rows_sl])
        pltpu.sync_copy(zeros_hbm.at[rows_sl], acc_sh.at[rows_sl])
        plsc.subcore_barrier()
        _edge_loop(NB_COL, src_hbm.at[s], dst_hbm.at[s], table_sh,
                   isrc_r, idst_r, (r0, r1), (sg0, sg1), (ss0, ss1),
                   (si0, si1, si2, si3), (sd0, sd1, sd2, sd3), acc_sh)
        plsc.subcore_barrier()
        pltpu.sync_copy(acc_sh.at[rows_sl], out_hbm.at[ch].at[rows_sl])
        if p + 1 < NCH // 2:
            plsc.subcore_barrier()


def _prop_kernel(NCH, FC):
    return pl.kernel(
        functools.partial(_prop_body, NCH),
        out_type=jax.ShapeDtypeStruct((NCH, NACC, FC), jnp.float32),
        mesh=_mesh(),
        compiler_params=_SC_PARAMS,
        scratch_types=(
            [pltpu.VMEM((4, BLK), jnp.int32)] * 2
            + [pltpu.VMEM((BLK, FC), jnp.float32)] * 2
            + [pltpu.SemaphoreType.DMA] * 12
            + [pltpu.VMEM_SHARED((NACC, FC), jnp.float32)] * 2
        ),
    )


BR = 1000


def _dinv_from(deg_ref):
    d = deg_ref[0, :, 0] + deg_ref[1, :, 0] + 1.0
    return lax.rsqrt(d)


def _elu(h):
    return jnp.where(h > 0.0, h, jnp.exp(h) - 1.0)


def _chunk_out(ref, m, nch):
    fc = m.shape[1] // nch
    for i in range(nch):
        ref[i] = m[:, i * fc:(i + 1) * fc]


def _unchunk(ref, nch):
    return jnp.concatenate([ref[i] for i in range(nch)], axis=1)


def _stageA_body(x_ref, w1_ref, a1_ref, c1_ref, w2_ref, a2_ref, c2_ref,
                 g1_ref, deg_ref, m1_ref):
    dinv = _dinv_from(deg_ref)
    h = jnp.dot(x_ref[...], w1_ref[...], precision=_HIGH)
    h = _elu(h * a1_ref[...] + c1_ref[...])
    h = jnp.dot(h, w2_ref[...], precision=_HIGH)
    h = _elu(h * a2_ref[...] + c2_ref[...])
    m = jnp.dot(h, g1_ref[...], precision=_HIGH)
    _chunk_out(m1_ref, m * dinv[:, None], 2)


def _stageB_body(p_ref, m1_ref, deg_ref, ag_ref, cg_ref, g2_ref, m2_ref):
    dinv = _dinv_from(deg_ref)
    ssum = _unchunk(p_ref, 2) + _unchunk(m1_ref, 2)
    h3 = jnp.maximum(ssum * dinv[:, None] * ag_ref[...] + cg_ref[...], 0.0)
    m2 = jnp.dot(h3, g2_ref[...], precision=_HIGH) * dinv[:, None]
    _chunk_out(m2_ref, m2, 2)


def _stageC_body(q_ref, m2_ref, deg_ref, ag_ref, cg_ref, e2d_ref, dec_ref, m3_ref):
    dinv = _dinv_from(deg_ref)
    ssum = _unchunk(q_ref, 2) + _unchunk(m2_ref, 2)
    h4 = ssum * dinv[:, None] * ag_ref[...] + cg_ref[...]
    h5 = jnp.dot(h4, e2d_ref[...], precision=_HIGH)
    m3 = jnp.dot(h5, dec_ref[...], precision=_HIGH) * dinv[:, None]
    _chunk_out(m3_ref, m3, 4)


def _stageD_body(r_ref, m3_ref, deg_ref, ad_ref, cd_ref, out_ref):
    dinv = _dinv_from(deg_ref)
    ssum = _unchunk(r_ref, 4) + _unchunk(m3_ref, 4)
    out_ref[...] = ssum * dinv[:, None] * ad_ref[...] + cd_ref[...]


def _row_spec(shape2):
    return pl.BlockSpec((BR,) + shape2[1:], lambda i: (i,) + (0,) * (len(shape2) - 1))


def _full_spec(shape):
    return pl.BlockSpec(shape, lambda i: (0,) * len(shape))


def _part_spec(nch, fc):
    return pl.BlockSpec((nch, BR, fc), lambda i: (0, i, 0))


def kernel(x, edge_index, params):
    p = params
    f32 = jnp.float32

    src = edge_index[0]
    dst = edge_index[1]
    pad = E_PAD - E
    src_p = jnp.concatenate([src, jnp.zeros((pad,), jnp.int32)])
    dst_p = jnp.concatenate([dst, jnp.full((pad,), N, jnp.int32)])
    dst_e = dst_p.reshape(NTILES, NB_EDGE, BLK)
    src_c = src_p.reshape(NSUB, NB_COL, BLK)
    dst_c = dst_p.reshape(NSUB, NB_COL, BLK)

    ones16 = jnp.ones((BLK, 16), f32)
    z16 = jnp.zeros((NACC, 16), f32)
    z32 = jnp.zeros((NACC, 32), f32)
    z64 = jnp.zeros((NACC, 64), f32)

    def row(v):
        return v.reshape(1, -1).astype(f32)

    a1 = p["bn1_g"] / jnp.sqrt(1.0 + 1e-3)
    c1 = p["enc1_b"] * a1 + p["bn1_b"]
    a2 = p["bn2_g"] / jnp.sqrt(1.0 + 1e-3)
    c2 = p["enc2_b"] * a2 + p["bn2_b"]
    ag1 = p["gc1bn_g"] / jnp.sqrt(1.0 + 1e-5)
    cg1 = p["gc1_b"] * ag1 + p["gc1bn_b"]
    ag2 = p["gc2bn_g"] / jnp.sqrt(1.0 + 1e-5)
    cg2 = p["gc2_b"] * ag2 + p["gc2bn_b"]
    ad = p["decbn_g"] / jnp.sqrt(1.0 + 1e-5)
    cd = p["dec_b"] * ad + p["decbn_b"]

    w1t = p["enc1_W"].T
    w2t = p["enc2_W"].T
    g1t = p["gc1_W"].T
    g2t = p["gc2_W"].T
    e2dt = p["e2d_W"].T
    dect = p["dec_W"].T

    deg = _deg_kernel()(dst_e, ones16, z16)

    m1 = pl.pallas_call(
        _stageA_body,
        grid=(N // BR,),
        in_specs=[
            _row_spec((N, D_IN)),
            _full_spec((D_IN, FH1)), _full_spec((1, FH1)), _full_spec((1, FH1)),
            _full_spec((FH1, FH2)), _full_spec((1, FH2)), _full_spec((1, FH2)),
            _full_spec((FH2, GH)),
            _part_spec(2, 16),
        ],
        out_specs=_part_spec(2, 64),
        out_shape=jax.ShapeDtypeStruct((2, NACC, 64), f32),
    )(x, w1t, row(a1), row(c1), w2t, row(a2), row(c2), g1t, deg)

    p1 = _prop_kernel(2, 64)(src_c, dst_c, m1, z64)

    m2 = pl.pallas_call(
        _stageB_body,
        grid=(N // BR,),
        in_specs=[
            _part_spec(2, 64), _part_spec(2, 64), _part_spec(2, 16),
            _full_spec((1, GH)), _full_spec((1, GH)), _full_spec((GH, LAT)),
        ],
        out_specs=_part_spec(2, 32),
        out_shape=jax.ShapeDtypeStruct((2, NACC, 32), f32),
    )(p1, m1, deg, row(ag1), row(cg1), g2t)

    p2 = _prop_kernel(2, 32)(src_c, dst_c, m2, z32)

    m3 = pl.pallas_call(
        _stageC_body,
        grid=(N // BR,),
        in_specs=[
            _part_spec(2, 32), _part_spec(2, 32), _part_spec(2, 16),
            _full_spec((1, LAT)), _full_spec((1, LAT)),
            _full_spec((LAT, LAT)), _full_spec((LAT, D_IN)),
        ],
        out_specs=_part_spec(4, 64),
        out_shape=jax.ShapeDtypeStruct((4, NACC, 64), f32),
    )(p2, m2, deg, row(ag2), row(cg2), e2dt, dect)

    p3 = _prop_kernel(4, 64)(src_c, dst_c, m3, z64)

    recon = pl.pallas_call(
        _stageD_body,
        grid=(N // BR,),
        in_specs=[
            _part_spec(4, 64), _part_spec(4, 64), _part_spec(2, 16),
            _full_spec((1, D_IN)), _full_spec((1, D_IN)),
        ],
        out_specs=_row_spec((N, D_IN)),
        out_shape=jax.ShapeDtypeStruct((N, D_IN), f32),
    )(p3, m3, deg, row(ad), row(cd))

    return recon


# speedup vs baseline: 16.9255x; 1.3872x over previous
import functools

import jax
import jax.numpy as jnp
from jax import lax
from jax.experimental import pallas as pl
from jax.experimental.pallas import tpu as pltpu
from jax.experimental.pallas import tpu_sc as plsc

N = 10000
E = 160000
D_IN = 256
FH1 = 512
FH2 = 256
GH = 128
LAT = 64

NTILES = 32
NSUB = 16
BLK = 128
E_PAD = 163840
NB_EDGE = E_PAD // (NTILES * BLK)
NB_COL = E_PAD // (NSUB * BLK)
NACC = 10112
RPT = NACC // NSUB

_HIGH = jax.lax.Precision.HIGHEST


def _mesh():
    return plsc.VectorSubcoreMesh(core_axis_name="c", subcore_axis_name="s")


_SC_PARAMS = pltpu.CompilerParams(use_tc_tiling_on_sc=False)


def _deg_body(dst_hbm, ones_hbm, zeros_hbm, out_hbm, idx_v, ones_v, sem, acc_sh):
    c = lax.axis_index("c")
    s = lax.axis_index("s")
    w = c * NSUB + s
    pltpu.sync_copy(dst_hbm.at[w], idx_v)
    pltpu.sync_copy(ones_hbm, ones_v)
    pltpu.sync_copy(zeros_hbm.at[pl.ds(s * RPT, RPT)], acc_sh.at[pl.ds(s * RPT, RPT)])
    plsc.subcore_barrier()

    def blk(j, carry):
        pltpu.async_copy(ones_v, acc_sh.at[idx_v.at[j]], sem, add=True).wait()
        return carry

    lax.fori_loop(0, NB_EDGE, blk, 0)
    plsc.subcore_barrier()
    pltpu.sync_copy(acc_sh.at[pl.ds(s * RPT, RPT)],
                    out_hbm.at[c].at[pl.ds(s * RPT, RPT)])


def _deg_kernel():
    return pl.kernel(
        _deg_body,
        out_type=jax.ShapeDtypeStruct((2, NACC, 16), jnp.float32),
        mesh=_mesh(),
        compiler_params=_SC_PARAMS,
        scratch_types=[
            pltpu.VMEM((NB_EDGE, BLK), jnp.int32),
            pltpu.VMEM((BLK, 16), jnp.float32),
            pltpu.SemaphoreType.DMA,
            pltpu.VMEM_SHARED((NACC, 16), jnp.float32),
        ],
    )


def _edge_loop(nb, slab_src, slab_dst, table_sh, isrc_r, idst_r,
               rows, semg, sems, semis, semid, acc_sh):
    def i_start(j, sl):
        pltpu.async_copy(slab_src.at[j], isrc_r.at[sl], semis[sl])
        pltpu.async_copy(slab_dst.at[j], idst_r.at[sl], semid[sl])

    def i_wait(j, sl):
        pltpu.make_async_copy(slab_src.at[j], isrc_r.at[sl], semis[sl]).wait()
        pltpu.make_async_copy(slab_dst.at[j], idst_r.at[sl], semid[sl]).wait()

    def g_start(sl4, b):
        pltpu.async_copy(table_sh.at[isrc_r.at[sl4]], rows[b], semg[b])

    def g_wait(sl4, b):
        pltpu.make_async_copy(table_sh.at[isrc_r.at[sl4]], rows[b], semg[b]).wait()

    def s_start(sl4, b):
        pltpu.async_copy(rows[b], acc_sh.at[idst_r.at[sl4]], sems[b], add=True)

    def s_wait(sl4, b):
        pltpu.make_async_copy(rows[b], acc_sh.at[idst_r.at[sl4]], sems[b]).wait()

    for j0 in range(4):
        i_start(j0, j0)
    i_wait(0, 0)
    g_start(0, 0)

    def group(g, carry):
        j0 = g * 4
        for b in range(4):
            j = j0 + b
            g_wait(b, b % 2)
            s_start(b, b % 2)

            @pl.when(j >= 1)
            def _():
                s_wait((b - 1) % 4, (b - 1) % 2)

            @pl.when(jnp.logical_and(j >= 1, j + 3 < nb))
            def _():
                i_start(j + 3, (b + 3) % 4)

            @pl.when(j + 1 < nb)
            def _():
                i_wait(j + 1, (b + 1) % 4)
                g_start((b + 1) % 4, (b + 1) % 2)

        return carry

    lax.fori_loop(0, nb // 4, group, 0)
    s_wait(3, 1)


def _prop_body(NCH, src_hbm, dst_hbm, mt_hbm, zeros_hbm, out_hbm,
               isrc_r, idst_r, r0, r1,
               sg0, sg1, ss0, ss1,
               si0, si1, si2, si3, sd0, sd1, sd2, sd3, table_sh, acc_sh):
    c = lax.axis_index("c")
    s = lax.axis_index("s")
    rows_sl = pl.ds(s * RPT, RPT)

    for p in range(NCH // 2):
        ch = c * (NCH // 2) + p
        pltpu.sync_copy(mt_hbm.at[ch].at[rows_sl], table_sh.at[rows_sl])
        pltpu.sync_copy(zeros_hbm.at[rows_sl], acc_sh.at[rows_sl])
        plsc.subcore_barrier()
        _edge_loop(NB_COL, src_hbm.at[s], dst_hbm.at[s], table_sh,
                   isrc_r, idst_r, (r0, r1), (sg0, sg1), (ss0, ss1),
                   (si0, si1, si2, si3), (sd0, sd1, sd2, sd3), acc_sh)
        plsc.subcore_barrier()
        pltpu.sync_copy(acc_sh.at[rows_sl], out_hbm.at[ch].at[rows_sl])
        if p + 1 < NCH // 2:
            plsc.subcore_barrier()


def _prop_kernel(NCH, FC):
    return pl.kernel(
        functools.partial(_prop_body, NCH),
        out_type=jax.ShapeDtypeStruct((NCH, NACC, FC), jnp.float32),
        mesh=_mesh(),
        compiler_params=_SC_PARAMS,
        scratch_types=(
            [pltpu.VMEM((4, BLK), jnp.int32)] * 2
            + [pltpu.VMEM((BLK, FC), jnp.float32)] * 2
            + [pltpu.SemaphoreType.DMA] * 12
            + [pltpu.VMEM_SHARED((NACC, FC), jnp.float32)] * 2
        ),
    )


BR = 1000


def _dinv_from(deg_ref):
    d = deg_ref[0, :, 0] + deg_ref[1, :, 0] + 1.0
    return lax.rsqrt(d)


def _elu(h):
    return jnp.where(h > 0.0, h, jnp.exp(h) - 1.0)


def _chunk_out(ref, m, nch):
    fc = m.shape[1] // nch
    for i in range(nch):
        ref[i] = m[:, i * fc:(i + 1) * fc]


def _unchunk(ref, nch):
    return jnp.concatenate([ref[i] for i in range(nch)], axis=1)


def _stageA_body(x_ref, w1_ref, a1_ref, c1_ref, w2_ref, a2_ref, c2_ref,
                 g1_ref, deg_ref, m1_ref):
    dinv = _dinv_from(deg_ref)
    h = jnp.dot(x_ref[...], w1_ref[...], precision=_HIGH)
    h = _elu(h * a1_ref[...] + c1_ref[...])
    h = jnp.dot(h, w2_ref[...], precision=_HIGH)
    h = _elu(h * a2_ref[...] + c2_ref[...])
    m = jnp.dot(h, g1_ref[...], precision=_HIGH)
    _chunk_out(m1_ref, m * dinv[:, None], 2)


def _stageB_body(p_ref, m1_ref, deg_ref, ag_ref, cg_ref, g2_ref, m2_ref):
    dinv = _dinv_from(deg_ref)
    ssum = _unchunk(p_ref, 2) + _unchunk(m1_ref, 2)
    h3 = jnp.maximum(ssum * dinv[:, None] * ag_ref[...] + cg_ref[...], 0.0)
    m2 = jnp.dot(h3, g2_ref[...], precision=_HIGH) * dinv[:, None]
    _chunk_out(m2_ref, m2, 2)


def _stageC_body(q_ref, m2_ref, deg_ref, ag_ref, cg_ref, e2d_ref, m3_ref):
    dinv = _dinv_from(deg_ref)
    ssum = _unchunk(q_ref, 2) + _unchunk(m2_ref, 2)
    h4 = ssum * dinv[:, None] * ag_ref[...] + cg_ref[...]
    h5 = jnp.dot(h4, e2d_ref[...], precision=_HIGH)
    _chunk_out(m3_ref, h5 * dinv[:, None], 2)


def _stageD_body(r_ref, m3_ref, deg_ref, dec_ref, ad_ref, cd_ref, out_ref):
    dinv = _dinv_from(deg_ref)
    ssum = _unchunk(r_ref, 2) + _unchunk(m3_ref, 2)
    rec = jnp.dot(ssum, dec_ref[...], precision=_HIGH)
    out_ref[...] = rec * dinv[:, None] * ad_ref[...] + cd_ref[...]


def _row_spec(shape2):
    return pl.BlockSpec((BR,) + shape2[1:], lambda i: (i,) + (0,) * (len(shape2) - 1))


def _full_spec(shape):
    return pl.BlockSpec(shape, lambda i: (0,) * len(shape))


def _part_spec(nch, fc):
    return pl.BlockSpec((nch, BR, fc), lambda i: (0, i, 0))


def kernel(x, edge_index, params):
    p = params
    f32 = jnp.float32

    src = edge_index[0]
    dst = edge_index[1]
    pad = E_PAD - E
    src_p = jnp.concatenate([src, jnp.zeros((pad,), jnp.int32)])
    dst_p = jnp.concatenate([dst, jnp.full((pad,), N, jnp.int32)])
    dst_e = dst_p.reshape(NTILES, NB_EDGE, BLK)
    src_c = src_p.reshape(NSUB, NB_COL, BLK)
    dst_c = dst_p.reshape(NSUB, NB_COL, BLK)

    ones16 = jnp.ones((BLK, 16), f32)
    z16 = jnp.zeros((NACC, 16), f32)
    z32 = jnp.zeros((NACC, 32), f32)
    z64 = jnp.zeros((NACC, 64), f32)

    def row(v):
        return v.reshape(1, -1).astype(f32)

    a1 = p["bn1_g"] / jnp.sqrt(1.0 + 1e-3)
    c1 = p["enc1_b"] * a1 + p["bn1_b"]
    a2 = p["bn2_g"] / jnp.sqrt(1.0 + 1e-3)
    c2 = p["enc2_b"] * a2 + p["bn2_b"]
    ag1 = p["gc1bn_g"] / jnp.sqrt(1.0 + 1e-5)
    cg1 = p["gc1_b"] * ag1 + p["gc1bn_b"]
    ag2 = p["gc2bn_g"] / jnp.sqrt(1.0 + 1e-5)
    cg2 = p["gc2_b"] * ag2 + p["gc2bn_b"]
    ad = p["decbn_g"] / jnp.sqrt(1.0 + 1e-5)
    cd = p["dec_b"] * ad + p["decbn_b"]

    w1t = p["enc1_W"].T
    w2t = p["enc2_W"].T
    g1t = p["gc1_W"].T
    g2t = p["gc2_W"].T
    e2dt = p["e2d_W"].T
    dect = p["dec_W"].T

    deg = _deg_kernel()(dst_e, ones16, z16)

    m1 = pl.pallas_call(
        _stageA_body,
        grid=(N // BR,),
        in_specs=[
            _row_spec((N, D_IN)),
            _full_spec((D_IN, FH1)), _full_spec((1, FH1)), _full_spec((1, FH1)),
            _full_spec((FH1, FH2)), _full_spec((1, FH2)), _full_spec((1, FH2)),
            _full_spec((FH2, GH)),
            _part_spec(2, 16),
        ],
        out_specs=_part_spec(2, 64),
        out_shape=jax.ShapeDtypeStruct((2, NACC, 64), f32),
    )(x, w1t, row(a1), row(c1), w2t, row(a2), row(c2), g1t, deg)

    p1 = _prop_kernel(2, 64)(src_c, dst_c, m1, z64)

    m2 = pl.pallas_call(
        _stageB_body,
        grid=(N // BR,),
        in_specs=[
            _part_spec(2, 64), _part_spec(2, 64), _part_spec(2, 16),
            _full_spec((1, GH)), _full_spec((1, GH)), _full_spec((GH, LAT)),
        ],
        out_specs=_part_spec(2, 32),
        out_shape=jax.ShapeDtypeStruct((2, NACC, 32), f32),
    )(p1, m1, deg, row(ag1), row(cg1), g2t)

    p2 = _prop_kernel(2, 32)(src_c, dst_c, m2, z32)

    m3 = pl.pallas_call(
        _stageC_body,
        grid=(N // BR,),
        in_specs=[
            _part_spec(2, 32), _part_spec(2, 32), _part_spec(2, 16),
            _full_spec((1, LAT)), _full_spec((1, LAT)),
            _full_spec((LAT, LAT)),
        ],
        out_specs=_part_spec(2, 32),
        out_shape=jax.ShapeDtypeStruct((2, NACC, 32), f32),
    )(p2, m2, deg, row(ag2), row(cg2), e2dt)

    p3 = _prop_kernel(2, 32)(src_c, dst_c, m3, z32)

    recon = pl.pallas_call(
        _stageD_body,
        grid=(N // BR,),
        in_specs=[
            _part_spec(2, 32), _part_spec(2, 32), _part_spec(2, 16),
            _full_spec((LAT, D_IN)),
            _full_spec((1, D_IN)), _full_spec((1, D_IN)),
        ],
        out_specs=_row_spec((N, D_IN)),
        out_shape=jax.ShapeDtypeStruct((N, D_IN), f32),
    )(p3, m3, deg, dect, row(ad), row(cd))

    return recon


# speedup vs baseline: 19.6956x; 1.1637x over previous
import functools

import jax
import jax.numpy as jnp
from jax import lax
from jax.experimental import pallas as pl
from jax.experimental.pallas import tpu as pltpu
from jax.experimental.pallas import tpu_sc as plsc

N = 10000
E = 160000
D_IN = 256
FH1 = 512
FH2 = 256
GH = 128
LAT = 64

NTILES = 32
NSUB = 16
BLK = 128
E_PAD = 163840
NB_EDGE = E_PAD // (NTILES * BLK)
NB_COL = E_PAD // (NSUB * BLK)
NACC = 10112
RPT = NACC // NSUB

_HIGH = jax.lax.Precision.DEFAULT


def _mesh():
    return plsc.VectorSubcoreMesh(core_axis_name="c", subcore_axis_name="s")


_SC_PARAMS = pltpu.CompilerParams(use_tc_tiling_on_sc=False)


def _deg_body(dst_hbm, ones_hbm, zeros_hbm, out_hbm, idx_v, ones_v, sem, acc_sh):
    c = lax.axis_index("c")
    s = lax.axis_index("s")
    w = c * NSUB + s
    pltpu.sync_copy(dst_hbm.at[w], idx_v)
    pltpu.sync_copy(ones_hbm, ones_v)
    pltpu.sync_copy(zeros_hbm.at[pl.ds(s * RPT, RPT)], acc_sh.at[pl.ds(s * RPT, RPT)])
    plsc.subcore_barrier()

    def blk(j, carry):
        pltpu.async_copy(ones_v, acc_sh.at[idx_v.at[j]], sem, add=True).wait()
        return carry

    lax.fori_loop(0, NB_EDGE, blk, 0)
    plsc.subcore_barrier()
    pltpu.sync_copy(acc_sh.at[pl.ds(s * RPT, RPT)],
                    out_hbm.at[c].at[pl.ds(s * RPT, RPT)])


def _deg_kernel():
    return pl.kernel(
        _deg_body,
        out_type=jax.ShapeDtypeStruct((2, NACC, 16), jnp.float32),
        mesh=_mesh(),
        compiler_params=_SC_PARAMS,
        scratch_types=[
            pltpu.VMEM((NB_EDGE, BLK), jnp.int32),
            pltpu.VMEM((BLK, 16), jnp.float32),
            pltpu.SemaphoreType.DMA,
            pltpu.VMEM_SHARED((NACC, 16), jnp.float32),
        ],
    )


def _edge_loop(nb, slab_src, slab_dst, table_sh, isrc_r, idst_r,
               rows, semg, sems, semis, semid, acc_sh):
    def i_start(j, sl):
        pltpu.async_copy(slab_src.at[j], isrc_r.at[sl], semis[sl])
        pltpu.async_copy(slab_dst.at[j], idst_r.at[sl], semid[sl])

    def i_wait(j, sl):
        pltpu.make_async_copy(slab_src.at[j], isrc_r.at[sl], semis[sl]).wait()
        pltpu.make_async_copy(slab_dst.at[j], idst_r.at[sl], semid[sl]).wait()

    def g_start(sl4, b):
        pltpu.async_copy(table_sh.at[isrc_r.at[sl4]], rows[b], semg[b])

    def g_wait(sl4, b):
        pltpu.make_async_copy(table_sh.at[isrc_r.at[sl4]], rows[b], semg[b]).wait()

    def s_start(sl4, b):
        pltpu.async_copy(rows[b], acc_sh.at[idst_r.at[sl4]], sems[b], add=True)

    def s_wait(sl4, b):
        pltpu.make_async_copy(rows[b], acc_sh.at[idst_r.at[sl4]], sems[b]).wait()

    for j0 in range(4):
        i_start(j0, j0)
    i_wait(0, 0)
    g_start(0, 0)

    def group(g, carry):
        j0 = g * 4
        for b in range(4):
            j = j0 + b
            g_wait(b, b % 2)
            s_start(b, b % 2)

            @pl.when(j >= 1)
            def _():
                s_wait((b - 1) % 4, (b - 1) % 2)

            @pl.when(jnp.logical_and(j >= 1, j + 3 < nb))
            def _():
                i_start(j + 3, (b + 3) % 4)

            @pl.when(j + 1 < nb)
            def _():
                i_wait(j + 1, (b + 1) % 4)
                g_start((b + 1) % 4, (b + 1) % 2)

        return carry

    lax.fori_loop(0, nb // 4, group, 0)
    s_wait(3, 1)


def _prop_body(NCH, src_hbm, dst_hbm, mt_hbm, zeros_hbm, out_hbm,
               isrc_r, idst_r, r0, r1,
               sg0, sg1, ss0, ss1,
               si0, si1, si2, si3, sd0, sd1, sd2, sd3, table_sh, acc_sh):
    c = lax.axis_index("c")
    s = lax.axis_index("s")
    rows_sl = pl.ds(s * RPT, RPT)

    for p in range(NCH // 2):
        ch = c * (NCH // 2) + p
        pltpu.sync_copy(mt_hbm.at[ch].at[rows_sl], table_sh.at[rows_sl])
        pltpu.sync_copy(zeros_hbm.at[rows_sl], acc_sh.at[rows_sl])
        plsc.subcore_barrier()
        _edge_loop(NB_COL, src_hbm.at[s], dst_hbm.at[s], table_sh,
                   isrc_r, idst_r, (r0, r1), (sg0, sg1), (ss0, ss1),
                   (si0, si1, si2, si3), (sd0, sd1, sd2, sd3), acc_sh)
        plsc.subcore_barrier()
        pltpu.sync_copy(acc_sh.at[rows_sl], out_hbm.at[ch].at[rows_sl])
        if p + 1 < NCH // 2:
            plsc.subcore_barrier()


def _prop_kernel(NCH, FC):
    return pl.kernel(
        functools.partial(_prop_body, NCH),
        out_type=jax.ShapeDtypeStruct((NCH, NACC, FC), jnp.float32),
        mesh=_mesh(),
        compiler_params=_SC_PARAMS,
        scratch_types=(
            [pltpu.VMEM((4, BLK), jnp.int32)] * 2
            + [pltpu.VMEM((BLK, FC), jnp.float32)] * 2
            + [pltpu.SemaphoreType.DMA] * 12
            + [pltpu.VMEM_SHARED((NACC, FC), jnp.float32)] * 2
        ),
    )


BR = 1000


def _dinv_from(deg_ref):
    d = deg_ref[0, :, 0] + deg_ref[1, :, 0] + 1.0
    return lax.rsqrt(d)


def _elu(h):
    return jnp.where(h > 0.0, h, jnp.exp(h) - 1.0)


def _chunk_out(ref, m, nch):
    fc = m.shape[1] // nch
    for i in range(nch):
        ref[i] = m[:, i * fc:(i + 1) * fc]


def _unchunk(ref, nch):
    return jnp.concatenate([ref[i] for i in range(nch)], axis=1)


def _stageA_body(x_ref, w1_ref, a1_ref, c1_ref, w2_ref, a2_ref, c2_ref,
                 g1_ref, deg_ref, m1_ref):
    dinv = _dinv_from(deg_ref)
    h = jnp.dot(x_ref[...], w1_ref[...], precision=_HIGH)
    h = _elu(h * a1_ref[...] + c1_ref[...])
    h = jnp.dot(h, w2_ref[...], precision=_HIGH)
    h = _elu(h * a2_ref[...] + c2_ref[...])
    m = jnp.dot(h, g1_ref[...], precision=_HIGH)
    _chunk_out(m1_ref, m * dinv[:, None], 2)


def _stageB_body(p_ref, m1_ref, deg_ref, ag_ref, cg_ref, g2_ref, m2_ref):
    dinv = _dinv_from(deg_ref)
    ssum = _unchunk(p_ref, 2) + _unchunk(m1_ref, 2)
    h3 = jnp.maximum(ssum * dinv[:, None] * ag_ref[...] + cg_ref[...], 0.0)
    m2 = jnp.dot(h3, g2_ref[...], precision=_HIGH) * dinv[:, None]
    _chunk_out(m2_ref, m2, 2)


def _stageC_body(q_ref, m2_ref, deg_ref, ag_ref, cg_ref, e2d_ref, m3_ref):
    dinv = _dinv_from(deg_ref)
    ssum = _unchunk(q_ref, 2) + _unchunk(m2_ref, 2)
    h4 = ssum * dinv[:, None] * ag_ref[...] + cg_ref[...]
    h5 = jnp.dot(h4, e2d_ref[...], precision=_HIGH)
    _chunk_out(m3_ref, h5 * dinv[:, None], 2)


def _stageD_body(r_ref, m3_ref, deg_ref, dec_ref, ad_ref, cd_ref, out_ref):
    dinv = _dinv_from(deg_ref)
    ssum = _unchunk(r_ref, 2) + _unchunk(m3_ref, 2)
    rec = jnp.dot(ssum, dec_ref[...], precision=_HIGH)
    out_ref[...] = rec * dinv[:, None] * ad_ref[...] + cd_ref[...]


def _row_spec(shape2):
    return pl.BlockSpec((BR,) + shape2[1:], lambda i: (i,) + (0,) * (len(shape2) - 1))


def _full_spec(shape):
    return pl.BlockSpec(shape, lambda i: (0,) * len(shape))


def _part_spec(nch, fc):
    return pl.BlockSpec((nch, BR, fc), lambda i: (0, i, 0))


def kernel(x, edge_index, params):
    p = params
    f32 = jnp.float32

    src = edge_index[0]
    dst = edge_index[1]
    pad = E_PAD - E
    src_p = jnp.concatenate([src, jnp.zeros((pad,), jnp.int32)])
    dst_p = jnp.concatenate([dst, jnp.full((pad,), N, jnp.int32)])
    dst_e = dst_p.reshape(NTILES, NB_EDGE, BLK)
    src_c = src_p.reshape(NSUB, NB_COL, BLK)
    dst_c = dst_p.reshape(NSUB, NB_COL, BLK)

    ones16 = jnp.ones((BLK, 16), f32)
    z16 = jnp.zeros((NACC, 16), f32)
    z32 = jnp.zeros((NACC, 32), f32)
    z64 = jnp.zeros((NACC, 64), f32)

    def row(v):
        return v.reshape(1, -1).astype(f32)

    a1 = p["bn1_g"] / jnp.sqrt(1.0 + 1e-3)
    c1 = p["enc1_b"] * a1 + p["bn1_b"]
    a2 = p["bn2_g"] / jnp.sqrt(1.0 + 1e-3)
    c2 = p["enc2_b"] * a2 + p["bn2_b"]
    ag1 = p["gc1bn_g"] / jnp.sqrt(1.0 + 1e-5)
    cg1 = p["gc1_b"] * ag1 + p["gc1bn_b"]
    ag2 = p["gc2bn_g"] / jnp.sqrt(1.0 + 1e-5)
    cg2 = p["gc2_b"] * ag2 + p["gc2bn_b"]
    ad = p["decbn_g"] / jnp.sqrt(1.0 + 1e-5)
    cd = p["dec_b"] * ad + p["decbn_b"]

    w1t = p["enc1_W"].T
    w2t = p["enc2_W"].T
    g1t = p["gc1_W"].T
    g2t = p["gc2_W"].T
    e2dt = p["e2d_W"].T
    dect = p["dec_W"].T

    deg = _deg_kernel()(dst_e, ones16, z16)

    m1 = pl.pallas_call(
        _stageA_body,
        grid=(N // BR,),
        in_specs=[
            _row_spec((N, D_IN)),
            _full_spec((D_IN, FH1)), _full_spec((1, FH1)), _full_spec((1, FH1)),
            _full_spec((FH1, FH2)), _full_spec((1, FH2)), _full_spec((1, FH2)),
            _full_spec((FH2, GH)),
            _part_spec(2, 16),
        ],
        out_specs=_part_spec(2, 64),
        out_shape=jax.ShapeDtypeStruct((2, NACC, 64), f32),
    )(x, w1t, row(a1), row(c1), w2t, row(a2), row(c2), g1t, deg)

    p1 = _prop_kernel(2, 64)(src_c, dst_c, m1, z64)

    m2 = pl.pallas_call(
        _stageB_body,
        grid=(N // BR,),
        in_specs=[
            _part_spec(2, 64), _part_spec(2, 64), _part_spec(2, 16),
            _full_spec((1, GH)), _full_spec((1, GH)), _full_spec((GH, LAT)),
        ],
        out_specs=_part_spec(2, 32),
        out_shape=jax.ShapeDtypeStruct((2, NACC, 32), f32),
    )(p1, m1, deg, row(ag1), row(cg1), g2t)

    p2 = _prop_kernel(2, 32)(src_c, dst_c, m2, z32)

    m3 = pl.pallas_call(
        _stageC_body,
        grid=(N // BR,),
        in_specs=[
            _part_spec(2, 32), _part_spec(2, 32), _part_spec(2, 16),
            _full_spec((1, LAT)), _full_spec((1, LAT)),
            _full_spec((LAT, LAT)),
        ],
        out_specs=_part_spec(2, 32),
        out_shape=jax.ShapeDtypeStruct((2, NACC, 32), f32),
    )(p2, m2, deg, row(ag2), row(cg2), e2dt)

    p3 = _prop_kernel(2, 32)(src_c, dst_c, m3, z32)

    recon = pl.pallas_call(
        _stageD_body,
        grid=(N // BR,),
        in_specs=[
            _part_spec(2, 32), _part_spec(2, 32), _part_spec(2, 16),
            _full_spec((LAT, D_IN)),
            _full_spec((1, D_IN)), _full_spec((1, D_IN)),
        ],
        out_specs=_row_spec((N, D_IN)),
        out_shape=jax.ShapeDtypeStruct((N, D_IN), f32),
    )(p3, m3, deg, dect, row(ad), row(cd))

    return recon


# speedup vs baseline: 20.0838x; 1.0197x over previous
import functools

import jax
import jax.numpy as jnp
from jax import lax
from jax.experimental import pallas as pl
from jax.experimental.pallas import tpu as pltpu
from jax.experimental.pallas import tpu_sc as plsc

N = 10000
E = 160000
D_IN = 256
FH1 = 512
FH2 = 256
GH = 128
LAT = 64

NTILES = 32
NSUB = 16
BLK = 128
E_PAD = 163840
NB_EDGE = E_PAD // (NTILES * BLK)
NB_COL = E_PAD // (NSUB * BLK)
NACC = 10112
RPT = NACC // NSUB

_HIGH = jax.lax.Precision.DEFAULT


def _mesh():
    return plsc.VectorSubcoreMesh(core_axis_name="c", subcore_axis_name="s")


_SC_PARAMS = pltpu.CompilerParams(use_tc_tiling_on_sc=False)


def _deg_body(dst_hbm, ones_hbm, zeros_hbm, out_hbm, idx_v, ones_v, sem, acc_sh):
    c = lax.axis_index("c")
    s = lax.axis_index("s")
    w = c * NSUB + s
    pltpu.sync_copy(dst_hbm.at[w], idx_v)
    pltpu.sync_copy(ones_hbm, ones_v)
    pltpu.sync_copy(zeros_hbm.at[pl.ds(s * RPT, RPT)], acc_sh.at[pl.ds(s * RPT, RPT)])
    plsc.subcore_barrier()

    def blk(j, carry):
        pltpu.async_copy(ones_v, acc_sh.at[idx_v.at[j]], sem, add=True).wait()
        return carry

    lax.fori_loop(0, NB_EDGE, blk, 0)
    plsc.subcore_barrier()
    pltpu.sync_copy(acc_sh.at[pl.ds(s * RPT, RPT)],
                    out_hbm.at[c].at[pl.ds(s * RPT, RPT)])


def _deg_kernel():
    return pl.kernel(
        _deg_body,
        out_type=jax.ShapeDtypeStruct((2, NACC, 16), jnp.float32),
        mesh=_mesh(),
        compiler_params=_SC_PARAMS,
        scratch_types=[
            pltpu.VMEM((NB_EDGE, BLK), jnp.int32),
            pltpu.VMEM((BLK, 16), jnp.float32),
            pltpu.SemaphoreType.DMA,
            pltpu.VMEM_SHARED((NACC, 16), jnp.float32),
        ],
    )


def _edge_loop(nb, slab_src, slab_dst, table_sh, isrc_r, idst_r,
               rows, semg, sems, semis, semid, acc_sh):
    def i_start(j, sl):
        pltpu.async_copy(slab_src.at[j], isrc_r.at[sl], semis[sl])
        pltpu.async_copy(slab_dst.at[j], idst_r.at[sl], semid[sl])

    def i_wait(j, sl):
        pltpu.make_async_copy(slab_src.at[j], isrc_r.at[sl], semis[sl]).wait()
        pltpu.make_async_copy(slab_dst.at[j], idst_r.at[sl], semid[sl]).wait()

    def g_start(sl4, b):
        pltpu.async_copy(table_sh.at[isrc_r.at[sl4]], rows[b], semg[b])

    def g_wait(sl4, b):
        pltpu.make_async_copy(table_sh.at[isrc_r.at[sl4]], rows[b], semg[b]).wait()

    def s_start(sl4, b):
        pltpu.async_copy(rows[b], acc_sh.at[idst_r.at[sl4]], sems[b], add=True)

    def s_wait(sl4, b):
        pltpu.make_async_copy(rows[b], acc_sh.at[idst_r.at[sl4]], sems[b]).wait()

    for j0 in range(4):
        i_start(j0, j0)
    i_wait(0, 0)
    g_start(0, 0)

    def group(g, carry):
        j0 = g * 4
        for b in range(4):
            j = j0 + b
            g_wait(b, b % 2)
            s_start(b, b % 2)

            @pl.when(j >= 1)
            def _():
                s_wait((b - 1) % 4, (b - 1) % 2)

            @pl.when(jnp.logical_and(j >= 1, j + 3 < nb))
            def _():
                i_start(j + 3, (b + 3) % 4)

            @pl.when(j + 1 < nb)
            def _():
                i_wait(j + 1, (b + 1) % 4)
                g_start((b + 1) % 4, (b + 1) % 2)

        return carry

    lax.fori_loop(0, nb // 4, group, 0)
    s_wait(3, 1)


def _prop_body(NCH, src_hbm, dst_hbm, mt_hbm, zeros_hbm, out_hbm,
               isrc_r, idst_r, r0, r1,
               sg0, sg1, ss0, ss1,
               si0, si1, si2, si3, sd0, sd1, sd2, sd3, table_sh, acc_sh):
    c = lax.axis_index("c")
    s = lax.axis_index("s")
    rows_sl = pl.ds(s * RPT, RPT)

    for p in range(NCH // 2):
        ch = c * (NCH // 2) + p
        pltpu.sync_copy(mt_hbm.at[ch].at[rows_sl], table_sh.at[rows_sl])
        pltpu.sync_copy(zeros_hbm.at[rows_sl], acc_sh.at[rows_sl])
        plsc.subcore_barrier()
        _edge_loop(NB_COL, src_hbm.at[s], dst_hbm.at[s], table_sh,
                   isrc_r, idst_r, (r0, r1), (sg0, sg1), (ss0, ss1),
                   (si0, si1, si2, si3), (sd0, sd1, sd2, sd3), acc_sh)
        plsc.subcore_barrier()
        pltpu.sync_copy(acc_sh.at[rows_sl], out_hbm.at[ch].at[rows_sl])
        if p + 1 < NCH // 2:
            plsc.subcore_barrier()


def _prop_kernel(NCH, FC):
    return pl.kernel(
        functools.partial(_prop_body, NCH),
        out_type=jax.ShapeDtypeStruct((NCH, NACC, FC), jnp.float32),
        mesh=_mesh(),
        compiler_params=_SC_PARAMS,
        scratch_types=(
            [pltpu.VMEM((4, BLK), jnp.int32)] * 2
            + [pltpu.VMEM((BLK, FC), jnp.float32)] * 2
            + [pltpu.SemaphoreType.DMA] * 12
            + [pltpu.VMEM_SHARED((NACC, FC), jnp.float32)] * 2
        ),
    )


BR = 2000


def _dinv_from(deg_ref):
    d = deg_ref[0, :, 0] + deg_ref[1, :, 0] + 1.0
    return lax.rsqrt(d)


def _elu(h):
    return jnp.where(h > 0.0, h, jnp.exp(h) - 1.0)


def _chunk_out(ref, m, nch):
    fc = m.shape[1] // nch
    for i in range(nch):
        ref[i] = m[:, i * fc:(i + 1) * fc]


def _unchunk(ref, nch):
    return jnp.concatenate([ref[i] for i in range(nch)], axis=1)


def _stageA_body(x_ref, w1_ref, a1_ref, c1_ref, w2_ref, a2_ref, c2_ref,
                 g1_ref, deg_ref, m1_ref):
    dinv = _dinv_from(deg_ref)
    h = jnp.dot(x_ref[...], w1_ref[...], precision=_HIGH)
    h = _elu(h * a1_ref[...] + c1_ref[...])
    h = jnp.dot(h, w2_ref[...], precision=_HIGH)
    h = _elu(h * a2_ref[...] + c2_ref[...])
    m = jnp.dot(h, g1_ref[...], precision=_HIGH)
    _chunk_out(m1_ref, m * dinv[:, None], 2)


def _stageB_body(p_ref, m1_ref, deg_ref, ag_ref, cg_ref, g2_ref, m2_ref):
    dinv = _dinv_from(deg_ref)
    ssum = _unchunk(p_ref, 2) + _unchunk(m1_ref, 2)
    h3 = jnp.maximum(ssum * dinv[:, None] * ag_ref[...] + cg_ref[...], 0.0)
    m2 = jnp.dot(h3, g2_ref[...], precision=_HIGH) * dinv[:, None]
    _chunk_out(m2_ref, m2, 2)


def _stageC_body(q_ref, m2_ref, deg_ref, ag_ref, cg_ref, e2d_ref, m3_ref):
    dinv = _dinv_from(deg_ref)
    ssum = _unchunk(q_ref, 2) + _unchunk(m2_ref, 2)
    h4 = ssum * dinv[:, None] * ag_ref[...] + cg_ref[...]
    h5 = jnp.dot(h4, e2d_ref[...], precision=_HIGH)
    _chunk_out(m3_ref, h5 * dinv[:, None], 2)


def _stageD_body(r_ref, m3_ref, deg_ref, dec_ref, ad_ref, cd_ref, out_ref):
    dinv = _dinv_from(deg_ref)
    ssum = _unchunk(r_ref, 2) + _unchunk(m3_ref, 2)
    rec = jnp.dot(ssum, dec_ref[...], precision=_HIGH)
    out_ref[...] = rec * dinv[:, None] * ad_ref[...] + cd_ref[...]


def _row_spec(shape2):
    return pl.BlockSpec((BR,) + shape2[1:], lambda i: (i,) + (0,) * (len(shape2) - 1))


def _full_spec(shape):
    return pl.BlockSpec(shape, lambda i: (0,) * len(shape))


def _part_spec(nch, fc):
    return pl.BlockSpec((nch, BR, fc), lambda i: (0, i, 0))


def kernel(x, edge_index, params):
    p = params
    f32 = jnp.float32

    src = edge_index[0]
    dst = edge_index[1]
    pad = E_PAD - E
    src_p = jnp.concatenate([src, jnp.zeros((pad,), jnp.int32)])
    dst_p = jnp.concatenate([dst, jnp.full((pad,), N, jnp.int32)])
    dst_e = dst_p.reshape(NTILES, NB_EDGE, BLK)
    src_c = src_p.reshape(NSUB, NB_COL, BLK)
    dst_c = dst_p.reshape(NSUB, NB_COL, BLK)

    ones16 = jnp.ones((BLK, 16), f32)
    z16 = jnp.zeros((NACC, 16), f32)
    z32 = jnp.zeros((NACC, 32), f32)
    z64 = jnp.zeros((NACC, 64), f32)

    def row(v):
        return v.reshape(1, -1).astype(f32)

    a1 = p["bn1_g"] / jnp.sqrt(1.0 + 1e-3)
    c1 = p["enc1_b"] * a1 + p["bn1_b"]
    a2 = p["bn2_g"] / jnp.sqrt(1.0 + 1e-3)
    c2 = p["enc2_b"] * a2 + p["bn2_b"]
    ag1 = p["gc1bn_g"] / jnp.sqrt(1.0 + 1e-5)
    cg1 = p["gc1_b"] * ag1 + p["gc1bn_b"]
    ag2 = p["gc2bn_g"] / jnp.sqrt(1.0 + 1e-5)
    cg2 = p["gc2_b"] * ag2 + p["gc2bn_b"]
    ad = p["decbn_g"] / jnp.sqrt(1.0 + 1e-5)
    cd = p["dec_b"] * ad + p["decbn_b"]

    w1t = p["enc1_W"].T
    w2t = p["enc2_W"].T
    g1t = p["gc1_W"].T
    g2t = p["gc2_W"].T
    e2dt = p["e2d_W"].T
    dect = p["dec_W"].T

    deg = _deg_kernel()(dst_e, ones16, z16)

    m1 = pl.pallas_call(
        _stageA_body,
        grid=(N // BR,),
        in_specs=[
            _row_spec((N, D_IN)),
            _full_spec((D_IN, FH1)), _full_spec((1, FH1)), _full_spec((1, FH1)),
            _full_spec((FH1, FH2)), _full_spec((1, FH2)), _full_spec((1, FH2)),
            _full_spec((FH2, GH)),
            _part_spec(2, 16),
        ],
        out_specs=_part_spec(2, 64),
        out_shape=jax.ShapeDtypeStruct((2, NACC, 64), f32),
    )(x, w1t, row(a1), row(c1), w2t, row(a2), row(c2), g1t, deg)

    p1 = _prop_kernel(2, 64)(src_c, dst_c, m1, z64)

    m2 = pl.pallas_call(
        _stageB_body,
        grid=(N // BR,),
        in_specs=[
            _part_spec(2, 64), _part_spec(2, 64), _part_spec(2, 16),
            _full_spec((1, GH)), _full_spec((1, GH)), _full_spec((GH, LAT)),
        ],
        out_specs=_part_spec(2, 32),
        out_shape=jax.ShapeDtypeStruct((2, NACC, 32), f32),
    )(p1, m1, deg, row(ag1), row(cg1), g2t)

    p2 = _prop_kernel(2, 32)(src_c, dst_c, m2, z32)

    m3 = pl.pallas_call(
        _stageC_body,
        grid=(N // BR,),
        in_specs=[
            _part_spec(2, 32), _part_spec(2, 32), _part_spec(2, 16),
            _full_spec((1, LAT)), _full_spec((1, LAT)),
            _full_spec((LAT, LAT)),
        ],
        out_specs=_part_spec(2, 32),
        out_shape=jax.ShapeDtypeStruct((2, NACC, 32), f32),
    )(p2, m2, deg, row(ag2), row(cg2), e2dt)

    p3 = _prop_kernel(2, 32)(src_c, dst_c, m3, z32)

    recon = pl.pallas_call(
        _stageD_body,
        grid=(N // BR,),
        in_specs=[
            _part_spec(2, 32), _part_spec(2, 32), _part_spec(2, 16),
            _full_spec((LAT, D_IN)),
            _full_spec((1, D_IN)), _full_spec((1, D_IN)),
        ],
        out_specs=_row_spec((N, D_IN)),
        out_shape=jax.ShapeDtypeStruct((N, D_IN), f32),
    )(p3, m3, deg, dect, row(ad), row(cd))

    return recon


# speedup vs baseline: 20.0855x; 1.0001x over previous
import functools

import jax
import jax.numpy as jnp
from jax import lax
from jax.experimental import pallas as pl
from jax.experimental.pallas import tpu as pltpu
from jax.experimental.pallas import tpu_sc as plsc

N = 10000
E = 160000
D_IN = 256
FH1 = 512
FH2 = 256
GH = 128
LAT = 64

NTILES = 32
NSUB = 16
BLK = 128
E_PAD = 163840
NB_EDGE = E_PAD // (NTILES * BLK)
NB_COL = E_PAD // (NSUB * BLK)
NACC = 10112
RPT = NACC // NSUB

_HIGH = jax.lax.Precision.DEFAULT


def _mesh():
    return plsc.VectorSubcoreMesh(core_axis_name="c", subcore_axis_name="s")


_SC_PARAMS = pltpu.CompilerParams(use_tc_tiling_on_sc=False)


def _deg_body(dst_hbm, ones_hbm, zeros_hbm, out_hbm, idx_v, ones_v, sem, acc_sh):
    c = lax.axis_index("c")
    s = lax.axis_index("s")
    w = c * NSUB + s
    pltpu.sync_copy(dst_hbm.at[w], idx_v)
    pltpu.sync_copy(ones_hbm, ones_v)
    pltpu.sync_copy(zeros_hbm.at[pl.ds(s * RPT, RPT)], acc_sh.at[pl.ds(s * RPT, RPT)])
    plsc.subcore_barrier()

    def blk(j, carry):
        pltpu.async_copy(ones_v, acc_sh.at[idx_v.at[j]], sem, add=True).wait()
        return carry

    lax.fori_loop(0, NB_EDGE, blk, 0)
    plsc.subcore_barrier()
    pltpu.sync_copy(acc_sh.at[pl.ds(s * RPT, RPT)],
                    out_hbm.at[c].at[pl.ds(s * RPT, RPT)])


def _deg_kernel():
    return pl.kernel(
        _deg_body,
        out_type=jax.ShapeDtypeStruct((2, NACC, 16), jnp.float32),
        mesh=_mesh(),
        compiler_params=_SC_PARAMS,
        scratch_types=[
            pltpu.VMEM((NB_EDGE, BLK), jnp.int32),
            pltpu.VMEM((BLK, 16), jnp.float32),
            pltpu.SemaphoreType.DMA,
            pltpu.VMEM_SHARED((NACC, 16), jnp.float32),
        ],
    )


def _edge_loop(nb, slab_src, slab_dst, table_sh, isrc_r, idst_r,
               rows, semg, sems, semis, semid, acc_sh):
    def i_start(j, sl):
        pltpu.async_copy(slab_src.at[j], isrc_r.at[sl], semis[sl])
        pltpu.async_copy(slab_dst.at[j], idst_r.at[sl], semid[sl])

    def i_wait(j, sl):
        pltpu.make_async_copy(slab_src.at[j], isrc_r.at[sl], semis[sl]).wait()
        pltpu.make_async_copy(slab_dst.at[j], idst_r.at[sl], semid[sl]).wait()

    def g_start(sl4, b):
        pltpu.async_copy(table_sh.at[isrc_r.at[sl4]], rows[b], semg[b])

    def g_wait(sl4, b):
        pltpu.make_async_copy(table_sh.at[isrc_r.at[sl4]], rows[b], semg[b]).wait()

    def s_start(sl4, b):
        pltpu.async_copy(rows[b], acc_sh.at[idst_r.at[sl4]], sems[b], add=True)

    def s_wait(sl4, b):
        pltpu.make_async_copy(rows[b], acc_sh.at[idst_r.at[sl4]], sems[b]).wait()

    for j0 in range(4):
        i_start(j0, j0)
    i_wait(0, 0)
    g_start(0, 0)

    def group(g, carry):
        j0 = g * 4
        for b in range(4):
            j = j0 + b
            g_wait(b, b % 2)
            s_start(b, b % 2)

            @pl.when(j >= 1)
            def _():
                s_wait((b - 1) % 4, (b - 1) % 2)

            @pl.when(jnp.logical_and(j >= 1, j + 3 < nb))
            def _():
                i_start(j + 3, (b + 3) % 4)

            @pl.when(j + 1 < nb)
            def _():
                i_wait(j + 1, (b + 1) % 4)
                g_start((b + 1) % 4, (b + 1) % 2)

        return carry

    lax.fori_loop(0, nb // 4, group, 0)
    s_wait(3, 1)


def _prop_body(NCH, src_hbm, dst_hbm, mt_hbm, zeros_hbm, out_hbm,
               isrc_r, idst_r, r0, r1,
               sg0, sg1, ss0, ss1,
               si0, si1, si2, si3, sd0, sd1, sd2, sd3, table_sh, acc_sh):
    c = lax.axis_index("c")
    s = lax.axis_index("s")
    rows_sl = pl.ds(s * RPT, RPT)

    for p in range(NCH // 2):
        ch = c * (NCH // 2) + p
        pltpu.sync_copy(mt_hbm.at[ch].at[rows_sl], table_sh.at[rows_sl])
        pltpu.sync_copy(zeros_hbm.at[rows_sl], acc_sh.at[rows_sl])
        plsc.subcore_barrier()
        _edge_loop(NB_COL, src_hbm.at[s], dst_hbm.at[s], table_sh,
                   isrc_r, idst_r, (r0, r1), (sg0, sg1), (ss0, ss1),
                   (si0, si1, si2, si3), (sd0, sd1, sd2, sd3), acc_sh)
        plsc.subcore_barrier()
        pltpu.sync_copy(acc_sh.at[rows_sl], out_hbm.at[ch].at[rows_sl])
        if p + 1 < NCH // 2:
            plsc.subcore_barrier()


def _prop_kernel(NCH, FC):
    return pl.kernel(
        functools.partial(_prop_body, NCH),
        out_type=jax.ShapeDtypeStruct((NCH, NACC, FC), jnp.float32),
        mesh=_mesh(),
        compiler_params=_SC_PARAMS,
        scratch_types=(
            [pltpu.VMEM((4, BLK), jnp.int32)] * 2
            + [pltpu.VMEM((BLK, FC), jnp.float32)] * 2
            + [pltpu.SemaphoreType.DMA] * 12
            + [pltpu.VMEM_SHARED((NACC, FC), jnp.float32)] * 2
        ),
    )


BR = 5000


def _dinv_from(deg_ref):
    d = deg_ref[0, :, 0] + deg_ref[1, :, 0] + 1.0
    return lax.rsqrt(d)


def _elu(h):
    return jnp.where(h > 0.0, h, jnp.exp(h) - 1.0)


def _chunk_out(ref, m, nch):
    fc = m.shape[1] // nch
    for i in range(nch):
        ref[i] = m[:, i * fc:(i + 1) * fc]


def _unchunk(ref, nch):
    return jnp.concatenate([ref[i] for i in range(nch)], axis=1)


def _stageA_body(x_ref, w1_ref, a1_ref, c1_ref, w2_ref, a2_ref, c2_ref,
                 g1_ref, deg_ref, m1_ref):
    dinv = _dinv_from(deg_ref)
    h = jnp.dot(x_ref[...], w1_ref[...], precision=_HIGH)
    h = _elu(h * a1_ref[...] + c1_ref[...])
    h = jnp.dot(h, w2_ref[...], precision=_HIGH)
    h = _elu(h * a2_ref[...] + c2_ref[...])
    m = jnp.dot(h, g1_ref[...], precision=_HIGH)
    _chunk_out(m1_ref, m * dinv[:, None], 2)


def _stageB_body(p_ref, m1_ref, deg_ref, ag_ref, cg_ref, g2_ref, m2_ref):
    dinv = _dinv_from(deg_ref)
    ssum = _unchunk(p_ref, 2) + _unchunk(m1_ref, 2)
    h3 = jnp.maximum(ssum * dinv[:, None] * ag_ref[...] + cg_ref[...], 0.0)
    m2 = jnp.dot(h3, g2_ref[...], precision=_HIGH) * dinv[:, None]
    _chunk_out(m2_ref, m2, 2)


def _stageC_body(q_ref, m2_ref, deg_ref, ag_ref, cg_ref, e2d_ref, m3_ref):
    dinv = _dinv_from(deg_ref)
    ssum = _unchunk(q_ref, 2) + _unchunk(m2_ref, 2)
    h4 = ssum * dinv[:, None] * ag_ref[...] + cg_ref[...]
    h5 = jnp.dot(h4, e2d_ref[...], precision=_HIGH)
    _chunk_out(m3_ref, h5 * dinv[:, None], 2)


def _stageD_body(r_ref, m3_ref, deg_ref, dec_ref, ad_ref, cd_ref, out_ref):
    dinv = _dinv_from(deg_ref)
    ssum = _unchunk(r_ref, 2) + _unchunk(m3_ref, 2)
    rec = jnp.dot(ssum, dec_ref[...], precision=_HIGH)
    out_ref[...] = rec * dinv[:, None] * ad_ref[...] + cd_ref[...]


def _row_spec(shape2):
    return pl.BlockSpec((BR,) + shape2[1:], lambda i: (i,) + (0,) * (len(shape2) - 1))


def _full_spec(shape):
    return pl.BlockSpec(shape, lambda i: (0,) * len(shape))


def _part_spec(nch, fc):
    return pl.BlockSpec((nch, BR, fc), lambda i: (0, i, 0))


def kernel(x, edge_index, params):
    p = params
    f32 = jnp.float32

    src = edge_index[0]
    dst = edge_index[1]
    pad = E_PAD - E
    src_p = jnp.concatenate([src, jnp.zeros((pad,), jnp.int32)])
    dst_p = jnp.concatenate([dst, jnp.full((pad,), N, jnp.int32)])
    dst_e = dst_p.reshape(NTILES, NB_EDGE, BLK)
    src_c = src_p.reshape(NSUB, NB_COL, BLK)
    dst_c = dst_p.reshape(NSUB, NB_COL, BLK)

    ones16 = jnp.ones((BLK, 16), f32)
    z16 = jnp.zeros((NACC, 16), f32)
    z32 = jnp.zeros((NACC, 32), f32)
    z64 = jnp.zeros((NACC, 64), f32)

    def row(v):
        return v.reshape(1, -1).astype(f32)

    a1 = p["bn1_g"] / jnp.sqrt(1.0 + 1e-3)
    c1 = p["enc1_b"] * a1 + p["bn1_b"]
    a2 = p["bn2_g"] / jnp.sqrt(1.0 + 1e-3)
    c2 = p["enc2_b"] * a2 + p["bn2_b"]
    ag1 = p["gc1bn_g"] / jnp.sqrt(1.0 + 1e-5)
    cg1 = p["gc1_b"] * ag1 + p["gc1bn_b"]
    ag2 = p["gc2bn_g"] / jnp.sqrt(1.0 + 1e-5)
    cg2 = p["gc2_b"] * ag2 + p["gc2bn_b"]
    ad = p["decbn_g"] / jnp.sqrt(1.0 + 1e-5)
    cd = p["dec_b"] * ad + p["decbn_b"]

    w1t = p["enc1_W"].T
    w2t = p["enc2_W"].T
    g1t = p["gc1_W"].T
    g2t = p["gc2_W"].T
    e2dt = p["e2d_W"].T
    dect = p["dec_W"].T

    deg = _deg_kernel()(dst_e, ones16, z16)

    m1 = pl.pallas_call(
        _stageA_body,
        grid=(N // BR,),
        in_specs=[
            _row_spec((N, D_IN)),
            _full_spec((D_IN, FH1)), _full_spec((1, FH1)), _full_spec((1, FH1)),
            _full_spec((FH1, FH2)), _full_spec((1, FH2)), _full_spec((1, FH2)),
            _full_spec((FH2, GH)),
            _part_spec(2, 16),
        ],
        out_specs=_part_spec(2, 64),
        out_shape=jax.ShapeDtypeStruct((2, NACC, 64), f32),
    )(x, w1t, row(a1), row(c1), w2t, row(a2), row(c2), g1t, deg)

    p1 = _prop_kernel(2, 64)(src_c, dst_c, m1, z64)

    m2 = pl.pallas_call(
        _stageB_body,
        grid=(N // BR,),
        in_specs=[
            _part_spec(2, 64), _part_spec(2, 64), _part_spec(2, 16),
            _full_spec((1, GH)), _full_spec((1, GH)), _full_spec((GH, LAT)),
        ],
        out_specs=_part_spec(2, 32),
        out_shape=jax.ShapeDtypeStruct((2, NACC, 32), f32),
    )(p1, m1, deg, row(ag1), row(cg1), g2t)

    p2 = _prop_kernel(2, 32)(src_c, dst_c, m2, z32)

    m3 = pl.pallas_call(
        _stageC_body,
        grid=(N // BR,),
        in_specs=[
            _part_spec(2, 32), _part_spec(2, 32), _part_spec(2, 16),
            _full_spec((1, LAT)), _full_spec((1, LAT)),
            _full_spec((LAT, LAT)),
        ],
        out_specs=_part_spec(2, 32),
        out_shape=jax.ShapeDtypeStruct((2, NACC, 32), f32),
    )(p2, m2, deg, row(ag2), row(cg2), e2dt)

    p3 = _prop_kernel(2, 32)(src_c, dst_c, m3, z32)

    recon = pl.pallas_call(
        _stageD_body,
        grid=(N // BR,),
        in_specs=[
            _part_spec(2, 32), _part_spec(2, 32), _part_spec(2, 16),
            _full_spec((LAT, D_IN)),
            _full_spec((1, D_IN)), _full_spec((1, D_IN)),
        ],
        out_specs=_row_spec((N, D_IN)),
        out_shape=jax.ShapeDtypeStruct((N, D_IN), f32),
    )(p3, m3, deg, dect, row(ad), row(cd))

    return recon
